# baseline (speedup 1.0000x reference)
import functools

import jax
import jax.numpy as jnp
from jax import lax
from jax.experimental import pallas as pl
from jax.experimental.pallas import tpu as pltpu
from jax.experimental.pallas import tpu_sc as plsc

B = 16384
N_CONT = 13
N_CAT = 26
VOCAB = 100000
EDIM = 16
OUT = N_CONT + N_CAT * EDIM
EPS = 1e-5

NC = 2
NS = 16
NW = NC * NS
RW = B // NW
R = 128
NCHUNK = RW // R
GID = 128
NGATH = (R * N_CAT) // GID

_mesh = plsc.VectorSubcoreMesh(core_axis_name="c", subcore_axis_name="s")
_cparams = pltpu.CompilerParams(use_tc_tiling_on_sc=False)


@functools.partial(
    pl.kernel,
    mesh=_mesh,
    compiler_params=_cparams,
    out_type=jax.ShapeDtypeStruct((NW, 64, 16), jnp.float32),
    scratch_types=[
        pltpu.VMEM((R * N_CAT,), jnp.int32),
        pltpu.VMEM((R * N_CAT, EDIM), jnp.float32),
        pltpu.VMEM((R, 16), jnp.float32),
        pltpu.VMEM((64, 16), jnp.float32),
        pltpu.SemaphoreType.DMA,
    ],
)
def _stats_kernel(idx_hbm, cont_hbm, tables_hbm, part_hbm,
                  idx_v, rows_v, cont_v, acc_v, sem):
    wid = lax.axis_index("s") * NC + lax.axis_index("c")
    base = wid * RW

    zeros = jnp.zeros((16,), jnp.float32)

    @pl.loop(0, 64)
    def _zero(i):
        acc_v[i, :] = zeros

    @pl.loop(0, NCHUNK)
    def _chunk(k):
        row0 = base + k * R
        pltpu.sync_copy(idx_hbm.at[pl.ds(row0 * N_CAT, R * N_CAT)], idx_v)
        pltpu.sync_copy(cont_hbm.at[pl.ds(row0, R)], cont_v)
        descs = [
            pltpu.async_copy(
                tables_hbm.at[idx_v.at[pl.ds(g * GID, GID)]],
                rows_v.at[pl.ds(g * GID, GID)],
                sem,
            )
            for g in range(NGATH)
        ]
        for d in descs:
            d.wait()

        for c in range(N_CAT):
            def emb_body(r, carry, c=c):
                s, q = carry
                v = rows_v[r * N_CAT + c, :]
                return s + v, q + v * v

            s, q = lax.fori_loop(0, R, emb_body, (acc_v[c, :], acc_v[N_CAT + c, :]))
            acc_v[c, :] = s
            acc_v[N_CAT + c, :] = q

        def cont_body(r, carry):
            s, q = carry
            v = cont_v[r, :]
            return s + v, q + v * v

        s, q = lax.fori_loop(0, R, cont_body, (acc_v[52, :], acc_v[53, :]))
        acc_v[52, :] = s
        acc_v[53, :] = q

    pltpu.sync_copy(acc_v, part_hbm.at[wid])


@functools.partial(
    pl.kernel,
    mesh=_mesh,
    compiler_params=_cparams,
    out_type=jax.ShapeDtypeStruct((B * OUT,), jnp.float32),
    scratch_types=[
        pltpu.VMEM((R * N_CAT,), jnp.int32),
        pltpu.VMEM((R * N_CAT, EDIM), jnp.float32),
        pltpu.VMEM((R, 16), jnp.float32),
        pltpu.VMEM((432,), jnp.float32),
        pltpu.VMEM((432,), jnp.float32),
        pltpu.VMEM((R * OUT + 16,), jnp.float32),
        pltpu.SemaphoreType.DMA,
    ],
)
def _apply_kernel(idx_hbm, cont_hbm, tables_hbm, scale_hbm, shift_hbm, out_hbm,
                  idx_v, rows_v, cont_v, scale_v, shift_v, out_v, sem):
    wid = lax.axis_index("s") * NC + lax.axis_index("c")
    base = wid * RW

    pltpu.sync_copy(scale_hbm, scale_v)
    pltpu.sync_copy(shift_hbm, shift_v)

    @pl.loop(0, NCHUNK)
    def _chunk(k):
        row0 = base + k * R
        pltpu.sync_copy(idx_hbm.at[pl.ds(row0 * N_CAT, R * N_CAT)], idx_v)
        pltpu.sync_copy(cont_hbm.at[pl.ds(row0, R)], cont_v)
        descs = [
            pltpu.async_copy(
                tables_hbm.at[idx_v.at[pl.ds(g * GID, GID)]],
                rows_v.at[pl.ds(g * GID, GID)],
                sem,
            )
            for g in range(NGATH)
        ]
        for d in descs:
            d.wait()

        sc = scale_v[pl.ds(N_CAT * EDIM, 16)]
        sh = shift_v[pl.ds(N_CAT * EDIM, 16)]

        def cont_body(r, _, sc=sc, sh=sh):
            v = cont_v[r, :]
            out_v[pl.ds(r * OUT + N_CAT * EDIM, 16)] = v * sc + sh
            return 0

        lax.fori_loop(0, R, cont_body, 0)

        for c in range(N_CAT):
            scc = scale_v[pl.ds(c * EDIM, 16)]
            shc = shift_v[pl.ds(c * EDIM, 16)]

            def emb_body(r, _, c=c, scc=scc, shc=shc):
                v = rows_v[r * N_CAT + c, :]
                out_v[pl.ds(r * OUT + c * EDIM, 16)] = v * scc + shc
                return 0

            lax.fori_loop(0, R, emb_body, 0)

        pltpu.sync_copy(out_v.at[pl.ds(0, R * OUT)],
                        out_hbm.at[pl.ds(row0 * OUT, R * OUT)])


def kernel(x_in, tables, bn_gamma, bn_beta):
    tables_flat = tables.reshape(N_CAT * VOCAB, EDIM)
    x_cat = x_in[:, N_CONT:].astype(jnp.int32)
    idx_flat = (x_cat + jnp.arange(N_CAT, dtype=jnp.int32) * VOCAB).reshape(-1)
    x_cont = x_in[:, :N_CONT]
    cont_pad = jnp.pad(x_cont, ((0, 0), (0, 16 - N_CONT)))

    parts = _stats_kernel(idx_flat, cont_pad, tables_flat)
    sums = parts.sum(axis=0)
    s_emb = sums[0:N_CAT].reshape(-1)
    q_emb = sums[N_CAT:2 * N_CAT].reshape(-1)
    s_all = jnp.concatenate([s_emb, sums[52]])
    q_all = jnp.concatenate([q_emb, sums[53]])
    mu = s_all / B
    var = jnp.maximum(q_all / B - mu * mu, 0.0)
    gamma_pad = jnp.concatenate([bn_gamma, jnp.zeros((3,), jnp.float32)])
    beta_pad = jnp.concatenate([bn_beta, jnp.zeros((3,), jnp.float32)])
    scale = gamma_pad / jnp.sqrt(var + EPS)
    shift = beta_pad - mu * scale

    out_flat = _apply_kernel(idx_flat, cont_pad, tables_flat, scale, shift)
    return out_flat.reshape(B, OUT)


# speedup vs baseline: 3.2236x; 3.2236x over previous
import functools

import jax
import jax.numpy as jnp
from jax import lax
from jax.experimental import pallas as pl
from jax.experimental.pallas import tpu as pltpu
from jax.experimental.pallas import tpu_sc as plsc

B = 16384
N_CONT = 13
N_CAT = 26
VOCAB = 100000
EDIM = 16
OUT = N_CONT + N_CAT * EDIM
EPS = 1e-5

NC = 2
NS = 16
NW = NC * NS
FPW = (N_CAT * EDIM) // NW
NV = B // 16
HB = B // 2

_mesh = plsc.VectorSubcoreMesh(core_axis_name="c", subcore_axis_name="s")
_cparams = pltpu.CompilerParams(use_tc_tiling_on_sc=True,
                                needs_layout_passes=False)

_MAGIC = 0x5F3759DF


def _rsqrt16(x):
    y = plsc.bitcast(jnp.int32(_MAGIC) - (plsc.bitcast(x, jnp.int32) >> 1),
                     jnp.float32)
    for _ in range(4):
        y = y * (1.5 - 0.5 * x * y * y)
    return y


def _batch_stats(load_fn):
    def body(i, carry):
        s, q = carry
        v = load_fn(i)
        return s + v, q + v * v

    z = jnp.zeros((16,), jnp.float32)
    s, q = lax.fori_loop(0, NV, body, (z, z))
    return jnp.sum(s), jnp.sum(q)


def _scale_shift(s, q, gamma_v, beta_v, f):
    fv = jnp.full((16,), f, jnp.int32)
    g = plsc.load_gather(gamma_v, [fv])
    b = plsc.load_gather(beta_v, [fv])
    mu = jnp.full((16,), s, jnp.float32) * (1.0 / B)
    msq = jnp.full((16,), q, jnp.float32) * (1.0 / B)
    var = jnp.maximum(msq - mu * mu, 0.0)
    scale = g * _rsqrt16(var + EPS)
    shift = b - mu * scale
    return scale, shift


@functools.partial(
    pl.kernel,
    mesh=_mesh,
    compiler_params=_cparams,
    out_type=jax.ShapeDtypeStruct((OUT, B), jnp.float32),
    scratch_types=[
        pltpu.VMEM((VOCAB,), jnp.float32),
        pltpu.VMEM((B,), jnp.float32),
        pltpu.VMEM((HB,), jnp.float32),
        pltpu.VMEM((OUT,), jnp.float32),
        pltpu.VMEM((OUT,), jnp.float32),
        pltpu.SemaphoreType.DMA,
    ],
)
def _cat_emb_head(tt_hbm, xt_hbm, gamma_hbm, beta_hbm, out_hbm,
                  row_v, idx_v, o_v, gamma_v, beta_v, sem):
    wid = lax.axis_index("s") * NC + lax.axis_index("c")

    pltpu.sync_copy(gamma_hbm, gamma_v)
    pltpu.sync_copy(beta_hbm, beta_v)

    for t in range(FPW):
        f = wid * FPW + t
        c = f // EDIM

        pltpu.sync_copy(xt_hbm.at[N_CONT + c], idx_v)

        @pl.loop(0, NV)
        def _cvt(i):
            v = idx_v[pl.ds(i * 16, 16)]
            idx_v[pl.ds(i * 16, 16)] = plsc.bitcast(v.astype(jnp.int32),
                                                    jnp.float32)

        pltpu.sync_copy(tt_hbm.at[f], row_v)

        def gat(i):
            ii = plsc.bitcast(idx_v[pl.ds(i * 16, 16)], jnp.int32)
            return plsc.load_gather(row_v, [ii])

        s, q = _batch_stats(gat)
        scale, shift = _scale_shift(s, q, gamma_v, beta_v, f)

        for h in range(2):
            @pl.loop(0, NV // 2)
            def _norm(i, h=h, scale=scale, shift=shift):
                g = gat(h * (NV // 2) + i)
                o_v[pl.ds(i * 16, 16)] = g * scale + shift

            pltpu.sync_copy(o_v, out_hbm.at[f, pl.ds(h * HB, HB)])

    @pl.when(wid < N_CONT)
    def _cont():
        r = wid
        fo = N_CAT * EDIM + r
        pltpu.sync_copy(xt_hbm.at[r], idx_v)

        def ld(i):
            return idx_v[pl.ds(i * 16, 16)]

        s, q = _batch_stats(ld)
        scale, shift = _scale_shift(s, q, gamma_v, beta_v, fo)

        for h in range(2):
            @pl.loop(0, NV // 2)
            def _norm(i, h=h, scale=scale, shift=shift):
                v = ld(h * (NV // 2) + i)
                o_v[pl.ds(i * 16, 16)] = v * scale + shift

            pltpu.sync_copy(o_v, out_hbm.at[fo, pl.ds(h * HB, HB)])


def kernel(x_in, tables, bn_gamma, bn_beta):
    tt = tables.transpose(0, 2, 1).reshape(N_CAT * EDIM, VOCAB)
    xt = x_in.T
    out_t = _cat_emb_head(tt, xt, bn_gamma, bn_beta)
    return out_t.T


# speedup vs baseline: 4.0751x; 1.2642x over previous
import functools

import jax
import jax.numpy as jnp
from jax import lax
from jax.experimental import pallas as pl
from jax.experimental.pallas import tpu as pltpu
from jax.experimental.pallas import tpu_sc as plsc

B = 16384
N_CONT = 13
N_CAT = 26
VOCAB = 100000
EDIM = 16
OUT = N_CONT + N_CAT * EDIM
EPS = 1e-5

NC = 2
NS = 16
NW = NC * NS
FPW = (N_CAT * EDIM) // NW
NV = B // 16
QB = B // 4
QV = NV // 4
UNROLL = 8

_mesh = plsc.VectorSubcoreMesh(core_axis_name="c", subcore_axis_name="s")
_cparams = pltpu.CompilerParams(use_tc_tiling_on_sc=True,
                                needs_layout_passes=False)

_MAGIC = 0x5F3759DF


def _rsqrt16(x):
    y = plsc.bitcast(jnp.int32(_MAGIC) - (plsc.bitcast(x, jnp.int32) >> 1),
                     jnp.float32)
    for _ in range(4):
        y = y * (1.5 - 0.5 * x * y * y)
    return y


def _batch_stats(load_fn):
    def body(i, carry):
        s, q = carry
        for k in range(UNROLL):
            v = load_fn(i * UNROLL + k)
            s = s + v
            q = q + v * v
        return s, q

    z = jnp.zeros((16,), jnp.float32)
    s, q = lax.fori_loop(0, NV // UNROLL, body, (z, z))
    return jnp.sum(s), jnp.sum(q)


def _scale_shift(s, q, gamma_v, beta_v, f):
    fv = jnp.full((16,), f, jnp.int32)
    g = plsc.load_gather(gamma_v, [fv])
    b = plsc.load_gather(beta_v, [fv])
    mu = jnp.full((16,), s, jnp.float32) * (1.0 / B)
    msq = jnp.full((16,), q, jnp.float32) * (1.0 / B)
    var = jnp.maximum(msq - mu * mu, 0.0)
    scale = g * _rsqrt16(var + EPS)
    shift = b - mu * scale
    return scale, shift


@functools.partial(
    pl.kernel,
    mesh=_mesh,
    compiler_params=_cparams,
    out_type=jax.ShapeDtypeStruct((OUT, B), jnp.float32),
    scratch_types=[
        pltpu.VMEM((VOCAB,), jnp.float32),
        pltpu.VMEM((B,), jnp.float32),
        pltpu.VMEM((2, QB), jnp.float32),
        pltpu.VMEM((OUT,), jnp.float32),
        pltpu.VMEM((OUT,), jnp.float32),
        pltpu.SemaphoreType.DMA,
        pltpu.SemaphoreType.DMA,
    ],
)
def _cat_emb_head(tt_hbm, xt_hbm, gamma_hbm, beta_hbm, out_hbm,
                  row_v, idx_v, o_v, gamma_v, beta_v, sem, osem):
    wid = lax.axis_index("s") * NC + lax.axis_index("c")

    pltpu.sync_copy(gamma_hbm, gamma_v)
    pltpu.sync_copy(beta_hbm, beta_v)

    def gat(i):
        ii = idx_v[pl.ds(i * 16, 16)].astype(jnp.int32)
        return plsc.load_gather(row_v, [ii])

    def norm_row(load_fn, scale, shift, f):
        pend = [None, None]
        for qq in range(4):
            slot = qq % 2
            if pend[slot] is not None:
                pend[slot].wait()

            @pl.loop(0, QV, step=UNROLL)
            def _n(i, qq=qq, slot=slot, scale=scale, shift=shift):
                for k in range(UNROLL):
                    v = load_fn(qq * QV + i + k)
                    o_v[slot, pl.ds((i + k) * 16, 16)] = v * scale + shift

            pend[slot] = pltpu.async_copy(
                o_v.at[slot], out_hbm.at[f, pl.ds(qq * QB, QB)], osem)
        pend[0].wait()
        pend[1].wait()

    for t in range(FPW):
        f = wid * FPW + t

        d_row = pltpu.async_copy(tt_hbm.at[f], row_v, sem)

        if t == 0:
            pltpu.sync_copy(xt_hbm.at[N_CONT + f // EDIM], idx_v)
        else:
            @pl.when(f % EDIM == 0)
            def _reload(f=f):
                pltpu.sync_copy(xt_hbm.at[N_CONT + f // EDIM], idx_v)

        d_row.wait()

        s, q = _batch_stats(gat)
        scale, shift = _scale_shift(s, q, gamma_v, beta_v, f)
        norm_row(gat, scale, shift, f)

    @pl.when(wid < N_CONT)
    def _cont():
        fo = N_CAT * EDIM + wid
        pltpu.sync_copy(xt_hbm.at[wid], idx_v)

        def ld(i):
            return idx_v[pl.ds(i * 16, 16)]

        s, q = _batch_stats(ld)
        scale, shift = _scale_shift(s, q, gamma_v, beta_v, fo)
        norm_row(ld, scale, shift, fo)


def kernel(x_in, tables, bn_gamma, bn_beta):
    tt = tables.transpose(0, 2, 1).reshape(N_CAT * EDIM, VOCAB)
    xt = x_in.T
    out_t = _cat_emb_head(tt, xt, bn_gamma, bn_beta)
    return out_t.T


# speedup vs baseline: 6.0285x; 1.4793x over previous
import functools

import jax
import jax.numpy as jnp
from jax import lax
from jax.experimental import pallas as pl
from jax.experimental.pallas import tpu as pltpu
from jax.experimental.pallas import tpu_sc as plsc

B = 16384
N_CONT = 13
N_CAT = 26
VOCAB = 100000
EDIM = 16
OUT = N_CONT + N_CAT * EDIM
EPS = 1e-5

NC = 2
NS = 16
NW = NC * NS
FPW = (N_CAT * EDIM) // NW
NV = B // 16
QB = B // 4
QV = NV // 4
UNROLL = 8
NACC = 4

_mesh = plsc.VectorSubcoreMesh(core_axis_name="c", subcore_axis_name="s")
_cparams = pltpu.CompilerParams(use_tc_tiling_on_sc=True,
                                needs_layout_passes=False)

_MAGIC = 0x5F3759DF


def _rsqrt16(x):
    y = plsc.bitcast(jnp.int32(_MAGIC) - (plsc.bitcast(x, jnp.int32) >> 1),
                     jnp.float32)
    for _ in range(4):
        y = y * (1.5 - 0.5 * x * y * y)
    return y


def _scale_shift(s, q, gamma_v, beta_v, f):
    fv = jnp.full((16,), f, jnp.int32)
    g = plsc.load_gather(gamma_v, [fv])
    b = plsc.load_gather(beta_v, [fv])
    mu = jnp.full((16,), s, jnp.float32) * (1.0 / B)
    msq = jnp.full((16,), q, jnp.float32) * (1.0 / B)
    var = jnp.maximum(msq - mu * mu, 0.0)
    scale = g * _rsqrt16(var + EPS)
    shift = b - mu * scale
    return scale, shift


def _stats(step_fn):
    z = jnp.zeros((16,), jnp.float32)

    def body(i, carry):
        acc = list(carry)
        for k in range(UNROLL):
            v = step_fn(i * UNROLL + k)
            a = k % NACC
            s, q = acc[2 * a], acc[2 * a + 1]
            acc[2 * a] = s + v
            acc[2 * a + 1] = q + v * v
        return tuple(acc)

    acc = lax.fori_loop(0, NV // UNROLL, body, (z,) * (2 * NACC))
    s = acc[0]
    q = acc[1]
    for a in range(1, NACC):
        s = s + acc[2 * a]
        q = q + acc[2 * a + 1]
    return jnp.sum(s), jnp.sum(q)


@functools.partial(
    pl.kernel,
    mesh=_mesh,
    compiler_params=_cparams,
    out_type=jax.ShapeDtypeStruct((OUT, B), jnp.float32),
    scratch_types=[
        pltpu.VMEM((VOCAB,), jnp.float32),
        pltpu.VMEM((B,), jnp.float32),
        pltpu.VMEM((2, QB), jnp.float32),
        pltpu.VMEM((OUT,), jnp.float32),
        pltpu.VMEM((OUT,), jnp.float32),
        pltpu.SemaphoreType.DMA,
        pltpu.SemaphoreType.DMA,
        pltpu.SemaphoreType.DMA,
    ],
)
def _cat_emb_head(tt_hbm, xt_hbm, gamma_hbm, beta_hbm, out_hbm,
                  row_v, g_v, o_v, gamma_v, beta_v, rsem, isem, osem):
    wid = lax.axis_index("s") * NC + lax.axis_index("c")

    pltpu.sync_copy(gamma_hbm, gamma_v)
    pltpu.sync_copy(beta_hbm, beta_v)

    def gather_store(i):
        ii = g_v[pl.ds(i * 16, 16)].astype(jnp.int32)
        v = plsc.load_gather(row_v, [ii])
        g_v[pl.ds(i * 16, 16)] = v
        return v

    def plain_load(i):
        return g_v[pl.ds(i * 16, 16)]

    def norm_row(scale, shift, f):
        pend = [None, None]
        for qq in range(4):
            slot = qq % 2
            if pend[slot] is not None:
                pend[slot].wait()

            @pl.loop(0, QV, step=UNROLL)
            def _n(i, qq=qq, slot=slot, scale=scale, shift=shift):
                for k in range(UNROLL):
                    v = plain_load(qq * QV + i + k)
                    o_v[slot, pl.ds((i + k) * 16, 16)] = v * scale + shift

            pend[slot] = pltpu.async_copy(
                o_v.at[slot], out_hbm.at[f, pl.ds(qq * QB, QB)], osem)
        return pend

    d_row = pltpu.async_copy(tt_hbm.at[wid * FPW], row_v, rsem)
    d_idx = pltpu.async_copy(xt_hbm.at[N_CONT + (wid * FPW) // EDIM], g_v, isem)
    pend_out = None

    for t in range(FPW):
        f = wid * FPW + t

        d_row.wait()
        d_idx.wait()

        s, q = _stats(gather_store)

        if t + 1 < FPW:
            d_row = pltpu.async_copy(tt_hbm.at[f + 1], row_v, rsem)
        scale, shift = _scale_shift(s, q, gamma_v, beta_v, f)

        if pend_out is not None:
            pend_out[0].wait()
            pend_out[1].wait()
        pend_out = norm_row(scale, shift, f)

        nxt = N_CONT + (f + 1) // EDIM if t + 1 < FPW else wid % N_CONT
        d_idx = pltpu.async_copy(xt_hbm.at[nxt], g_v, isem)

    d_idx.wait()
    pend_out[0].wait()
    pend_out[1].wait()

    @pl.when(wid < N_CONT)
    def _cont():
        fo = N_CAT * EDIM + wid
        s, q = _stats(plain_load)
        scale, shift = _scale_shift(s, q, gamma_v, beta_v, fo)
        pend = norm_row(scale, shift, fo)
        pend[0].wait()
        pend[1].wait()


def kernel(x_in, tables, bn_gamma, bn_beta):
    tt = tables.transpose(0, 2, 1).reshape(N_CAT * EDIM, VOCAB)
    xt = x_in.T
    out_t = _cat_emb_head(tt, xt, bn_gamma, bn_beta)
    return out_t.T


# speedup vs baseline: 9.1863x; 1.5238x over previous
import functools

import jax
import jax.numpy as jnp
from jax import lax
from jax.experimental import pallas as pl
from jax.experimental.pallas import tpu as pltpu
from jax.experimental.pallas import tpu_sc as plsc

B = 16384
N_CONT = 13
N_CAT = 26
VOCAB = 100000
EDIM = 16
OUT = N_CONT + N_CAT * EDIM
EPS = 1e-5

NC = 2
NS = 16
NW = NC * NS
FPW = (N_CAT * EDIM) // NW
NV = B // 16
QB = B // 4
QV = NV // 4
UNROLL = 8
NACC = 4

_mesh = plsc.VectorSubcoreMesh(core_axis_name="c", subcore_axis_name="s")
_cparams = pltpu.CompilerParams(use_tc_tiling_on_sc=True,
                                needs_layout_passes=False)

_MAGIC = 0x5F3759DF


def _rsqrt16(x):
    y = plsc.bitcast(jnp.int32(_MAGIC) - (plsc.bitcast(x, jnp.int32) >> 1),
                     jnp.float32)
    for _ in range(4):
        y = y * (1.5 - 0.5 * x * y * y)
    return y


def _scale_shift(s, q, gamma_v, beta_v, f):
    fv = jnp.full((16,), f, jnp.int32)
    g = plsc.load_gather(gamma_v, [fv])
    b = plsc.load_gather(beta_v, [fv])
    mu = jnp.full((16,), s, jnp.float32) * (1.0 / B)
    msq = jnp.full((16,), q, jnp.float32) * (1.0 / B)
    var = jnp.maximum(msq - mu * mu, 0.0)
    scale = g * _rsqrt16(var + EPS)
    shift = b - mu * scale
    return scale, shift


def _stats(block_fn):
    z = jnp.zeros((16,), jnp.float32)

    def body(i, carry):
        acc = list(carry)
        vals = block_fn(i * UNROLL)
        for k in range(UNROLL):
            v = vals[k]
            a = k % NACC
            acc[2 * a] = acc[2 * a] + v
            acc[2 * a + 1] = acc[2 * a + 1] + v * v
        return tuple(acc)

    acc = lax.fori_loop(0, NV // UNROLL, body, (z,) * (2 * NACC))
    s = acc[0]
    q = acc[1]
    for a in range(1, NACC):
        s = s + acc[2 * a]
        q = q + acc[2 * a + 1]
    return jnp.sum(s), jnp.sum(q)


@functools.partial(
    pl.kernel,
    mesh=_mesh,
    compiler_params=_cparams,
    out_type=jax.ShapeDtypeStruct((OUT, B), jnp.float32),
    scratch_types=[
        pltpu.VMEM((VOCAB,), jnp.float32),
        pltpu.VMEM((B,), jnp.float32),
        pltpu.VMEM((2, QB), jnp.float32),
        pltpu.VMEM((OUT,), jnp.float32),
        pltpu.VMEM((OUT,), jnp.float32),
        pltpu.SemaphoreType.DMA,
        pltpu.SemaphoreType.DMA,
        pltpu.SemaphoreType.DMA,
    ],
)
def _cat_emb_head(tt_hbm, xt_hbm, gamma_hbm, beta_hbm, out_hbm,
                  row_v, g_v, o_v, gamma_v, beta_v, rsem, isem, osem):
    wid = lax.axis_index("s") * NC + lax.axis_index("c")

    pltpu.sync_copy(gamma_hbm, gamma_v)
    pltpu.sync_copy(beta_hbm, beta_v)

    def gather_store_block(i0):
        codes = [g_v[pl.ds((i0 + k) * 16, 16)] for k in range(UNROLL)]
        iis = [v.astype(jnp.int32) for v in codes]
        vals = [plsc.load_gather(row_v, [ii]) for ii in iis]
        for k in range(UNROLL):
            g_v[pl.ds((i0 + k) * 16, 16)] = vals[k]
        return vals

    def load_block(i0):
        return [g_v[pl.ds((i0 + k) * 16, 16)] for k in range(UNROLL)]

    def norm_row(scale, shift, f):
        pend = [None, None]
        for qq in range(4):
            slot = qq % 2
            if pend[slot] is not None:
                pend[slot].wait()

            @pl.loop(0, QV, step=UNROLL)
            def _n(i, qq=qq, slot=slot, scale=scale, shift=shift):
                vs = [g_v[pl.ds((qq * QV + i + k) * 16, 16)]
                      for k in range(UNROLL)]
                for k in range(UNROLL):
                    o_v[slot, pl.ds((i + k) * 16, 16)] = vs[k] * scale + shift

            pend[slot] = pltpu.async_copy(
                o_v.at[slot], out_hbm.at[f, pl.ds(qq * QB, QB)], osem)
        return pend

    d_row = pltpu.async_copy(tt_hbm.at[wid * FPW], row_v, rsem)
    d_idx = pltpu.async_copy(xt_hbm.at[N_CONT + (wid * FPW) // EDIM], g_v, isem)
    pend_out = None

    for t in range(FPW):
        f = wid * FPW + t

        d_row.wait()
        d_idx.wait()

        s, q = _stats(gather_store_block)

        if t + 1 < FPW:
            d_row = pltpu.async_copy(tt_hbm.at[f + 1], row_v, rsem)
        scale, shift = _scale_shift(s, q, gamma_v, beta_v, f)

        if pend_out is not None:
            pend_out[0].wait()
            pend_out[1].wait()
        pend_out = norm_row(scale, shift, f)

        nxt = N_CONT + (f + 1) // EDIM if t + 1 < FPW else wid % N_CONT
        d_idx = pltpu.async_copy(xt_hbm.at[nxt], g_v, isem)

    d_idx.wait()
    pend_out[0].wait()
    pend_out[1].wait()

    @pl.when(wid < N_CONT)
    def _cont():
        fo = N_CAT * EDIM + wid
        s, q = _stats(load_block)
        scale, shift = _scale_shift(s, q, gamma_v, beta_v, fo)
        pend = norm_row(scale, shift, fo)
        pend[0].wait()
        pend[1].wait()


def kernel(x_in, tables, bn_gamma, bn_beta):
    tt = tables.transpose(0, 2, 1).reshape(N_CAT * EDIM, VOCAB)
    xt = x_in.T
    out_t = _cat_emb_head(tt, xt, bn_gamma, bn_beta)
    return out_t.T


# speedup vs baseline: 9.4521x; 1.0289x over previous
import functools

import jax
import jax.numpy as jnp
from jax import lax
from jax.experimental import pallas as pl
from jax.experimental.pallas import tpu as pltpu
from jax.experimental.pallas import tpu_sc as plsc

B = 16384
N_CONT = 13
N_CAT = 26
VOCAB = 100000
EDIM = 16
OUT = N_CONT + N_CAT * EDIM
EPS = 1e-5

NC = 2
NS = 16
NW = NC * NS
FPW = (N_CAT * EDIM) // NW
NV = B // 16
QB = B // 4
QV = NV // 4
UNROLL = 16
NACC = 4

_mesh = plsc.VectorSubcoreMesh(core_axis_name="c", subcore_axis_name="s")
_cparams = pltpu.CompilerParams(use_tc_tiling_on_sc=True,
                                needs_layout_passes=False)

_MAGIC = 0x5F3759DF


def _rsqrt16(x):
    y = plsc.bitcast(jnp.int32(_MAGIC) - (plsc.bitcast(x, jnp.int32) >> 1),
                     jnp.float32)
    for _ in range(4):
        y = y * (1.5 - 0.5 * x * y * y)
    return y


def _scale_shift(s, q, gamma_v, beta_v, f):
    fv = jnp.full((16,), f, jnp.int32)
    g = plsc.load_gather(gamma_v, [fv])
    b = plsc.load_gather(beta_v, [fv])
    mu = jnp.full((16,), s, jnp.float32) * (1.0 / B)
    msq = jnp.full((16,), q, jnp.float32) * (1.0 / B)
    var = jnp.maximum(msq - mu * mu, 0.0)
    scale = g * _rsqrt16(var + EPS)
    shift = b - mu * scale
    return scale, shift


def _stats(block_fn):
    z = jnp.zeros((16,), jnp.float32)

    def body(i, carry):
        acc = list(carry)
        vals = block_fn(i * UNROLL)
        for k in range(UNROLL):
            v = vals[k]
            a = k % NACC
            acc[2 * a] = acc[2 * a] + v
            acc[2 * a + 1] = acc[2 * a + 1] + v * v
        return tuple(acc)

    acc = lax.fori_loop(0, NV // UNROLL, body, (z,) * (2 * NACC))
    s = acc[0]
    q = acc[1]
    for a in range(1, NACC):
        s = s + acc[2 * a]
        q = q + acc[2 * a + 1]
    return jnp.sum(s), jnp.sum(q)


@functools.partial(
    pl.kernel,
    mesh=_mesh,
    compiler_params=_cparams,
    out_type=jax.ShapeDtypeStruct((OUT, B), jnp.float32),
    scratch_types=[
        pltpu.VMEM((VOCAB,), jnp.float32),
        pltpu.VMEM((B,), jnp.float32),
        pltpu.VMEM((2, QB), jnp.float32),
        pltpu.VMEM((OUT,), jnp.float32),
        pltpu.VMEM((OUT,), jnp.float32),
        pltpu.SemaphoreType.DMA,
        pltpu.SemaphoreType.DMA,
        pltpu.SemaphoreType.DMA,
    ],
)
def _cat_emb_head(tt_hbm, xt_hbm, gamma_hbm, beta_hbm, out_hbm,
                  row_v, g_v, o_v, gamma_v, beta_v, rsem, isem, osem):
    wid = lax.axis_index("s") * NC + lax.axis_index("c")

    pltpu.sync_copy(gamma_hbm, gamma_v)
    pltpu.sync_copy(beta_hbm, beta_v)

    def gather_store_block(i0):
        codes = [g_v[pl.ds((i0 + k) * 16, 16)] for k in range(UNROLL)]
        iis = [v.astype(jnp.int32) for v in codes]
        vals = [plsc.load_gather(row_v, [ii]) for ii in iis]
        for k in range(UNROLL):
            g_v[pl.ds((i0 + k) * 16, 16)] = vals[k]
        return vals

    def load_block(i0):
        return [g_v[pl.ds((i0 + k) * 16, 16)] for k in range(UNROLL)]

    def norm_row(scale, shift, f):
        pend = [None, None]
        for qq in range(4):
            slot = qq % 2
            if pend[slot] is not None:
                pend[slot].wait()

            @pl.loop(0, QV, step=UNROLL)
            def _n(i, qq=qq, slot=slot, scale=scale, shift=shift):
                vs = [g_v[pl.ds((qq * QV + i + k) * 16, 16)]
                      for k in range(UNROLL)]
                for k in range(UNROLL):
                    o_v[slot, pl.ds((i + k) * 16, 16)] = vs[k] * scale + shift

            pend[slot] = pltpu.async_copy(
                o_v.at[slot], out_hbm.at[f, pl.ds(qq * QB, QB)], osem)
        return pend

    d_row = pltpu.async_copy(tt_hbm.at[wid * FPW], row_v, rsem)
    d_idx = pltpu.async_copy(xt_hbm.at[N_CONT + (wid * FPW) // EDIM], g_v, isem)
    pend_out = None

    for t in range(FPW):
        f = wid * FPW + t

        d_row.wait()
        d_idx.wait()

        s, q = _stats(gather_store_block)

        if t + 1 < FPW:
            d_row = pltpu.async_copy(tt_hbm.at[f + 1], row_v, rsem)
        scale, shift = _scale_shift(s, q, gamma_v, beta_v, f)

        if pend_out is not None:
            pend_out[0].wait()
            pend_out[1].wait()
        pend_out = norm_row(scale, shift, f)

        nxt = N_CONT + (f + 1) // EDIM if t + 1 < FPW else wid % N_CONT
        d_idx = pltpu.async_copy(xt_hbm.at[nxt], g_v, isem)

    d_idx.wait()
    pend_out[0].wait()
    pend_out[1].wait()

    @pl.when(wid < N_CONT)
    def _cont():
        fo = N_CAT * EDIM + wid
        s, q = _stats(load_block)
        scale, shift = _scale_shift(s, q, gamma_v, beta_v, fo)
        pend = norm_row(scale, shift, fo)
        pend[0].wait()
        pend[1].wait()


def kernel(x_in, tables, bn_gamma, bn_beta):
    tt = tables.transpose(0, 2, 1).reshape(N_CAT * EDIM, VOCAB)
    xt = x_in.T
    out_t = _cat_emb_head(tt, xt, bn_gamma, bn_beta)
    return out_t.T


# speedup vs baseline: 9.8675x; 1.0439x over previous
import functools

import jax
import jax.numpy as jnp
from jax import lax
from jax.experimental import pallas as pl
from jax.experimental.pallas import tpu as pltpu
from jax.experimental.pallas import tpu_sc as plsc

B = 16384
N_CONT = 13
N_CAT = 26
VOCAB = 100000
EDIM = 16
OUT = N_CONT + N_CAT * EDIM
EPS = 1e-5

NC = 2
NS = 16
NW = NC * NS
FPW = (N_CAT * EDIM) // NW
NV = B // 16
UNROLL = 16
NACC = 4

_mesh = plsc.VectorSubcoreMesh(core_axis_name="c", subcore_axis_name="s")
_cparams = pltpu.CompilerParams(use_tc_tiling_on_sc=True,
                                needs_layout_passes=False)

_MAGIC = 0x5F3759DF


def _rsqrt16(x):
    y = plsc.bitcast(jnp.int32(_MAGIC) - (plsc.bitcast(x, jnp.int32) >> 1),
                     jnp.float32)
    for _ in range(4):
        y = y * (1.5 - 0.5 * x * y * y)
    return y


def _scale_shift(s, q, gamma_v, beta_v, f):
    fv = jnp.full((16,), f, jnp.int32)
    g = plsc.load_gather(gamma_v, [fv])
    b = plsc.load_gather(beta_v, [fv])
    mu = jnp.full((16,), s, jnp.float32) * (1.0 / B)
    msq = jnp.full((16,), q, jnp.float32) * (1.0 / B)
    var = jnp.maximum(msq - mu * mu, 0.0)
    scale = g * _rsqrt16(var + EPS)
    shift = b - mu * scale
    return scale, shift


def _stats(block_fn):
    z = jnp.zeros((16,), jnp.float32)

    def body(i, carry):
        acc = list(carry)
        vals = block_fn(i * UNROLL)
        for k in range(UNROLL):
            v = vals[k]
            a = k % NACC
            acc[2 * a] = acc[2 * a] + v
            acc[2 * a + 1] = acc[2 * a + 1] + v * v
        return tuple(acc)

    acc = lax.fori_loop(0, NV // UNROLL, body, (z,) * (2 * NACC))
    s = acc[0]
    q = acc[1]
    for a in range(1, NACC):
        s = s + acc[2 * a]
        q = q + acc[2 * a + 1]
    return jnp.sum(s), jnp.sum(q)


@functools.partial(
    pl.kernel,
    mesh=_mesh,
    compiler_params=_cparams,
    out_type=jax.ShapeDtypeStruct((OUT, B), jnp.float32),
    scratch_types=[
        pltpu.VMEM((VOCAB,), jnp.float32),
        pltpu.VMEM((B,), jnp.float32),
        pltpu.VMEM((OUT,), jnp.float32),
        pltpu.VMEM((OUT,), jnp.float32),
        pltpu.SemaphoreType.DMA,
        pltpu.SemaphoreType.DMA,
        pltpu.SemaphoreType.DMA,
    ],
)
def _cat_emb_head(tt_hbm, xt_hbm, gamma_hbm, beta_hbm, out_hbm,
                  row_v, g_v, gamma_v, beta_v, rsem, isem, osem):
    wid = lax.axis_index("s") * NC + lax.axis_index("c")

    pltpu.sync_copy(gamma_hbm, gamma_v)
    pltpu.sync_copy(beta_hbm, beta_v)

    def gather_store_block(i0):
        codes = [g_v[pl.ds((i0 + k) * 16, 16)] for k in range(UNROLL)]
        iis = [v.astype(jnp.int32) for v in codes]
        vals = [plsc.load_gather(row_v, [ii]) for ii in iis]
        for k in range(UNROLL):
            g_v[pl.ds((i0 + k) * 16, 16)] = vals[k]
        return vals

    def load_block(i0):
        return [g_v[pl.ds((i0 + k) * 16, 16)] for k in range(UNROLL)]

    def norm_inplace(scale, shift):
        @pl.loop(0, NV, step=UNROLL)
        def _n(i, scale=scale, shift=shift):
            vs = load_block(i)
            for k in range(UNROLL):
                g_v[pl.ds((i + k) * 16, 16)] = vs[k] * scale + shift

    d_row = pltpu.async_copy(tt_hbm.at[wid * FPW], row_v, rsem)
    d_idx = pltpu.async_copy(xt_hbm.at[N_CONT + (wid * FPW) // EDIM], g_v, isem)

    for t in range(FPW):
        f = wid * FPW + t

        d_row.wait()
        d_idx.wait()

        s, q = _stats(gather_store_block)

        if t + 1 < FPW:
            d_row = pltpu.async_copy(tt_hbm.at[f + 1], row_v, rsem)
        scale, shift = _scale_shift(s, q, gamma_v, beta_v, f)
        norm_inplace(scale, shift)

        pltpu.async_copy(g_v, out_hbm.at[f], osem).wait()

        nxt = N_CONT + (f + 1) // EDIM if t + 1 < FPW else wid % N_CONT
        d_idx = pltpu.async_copy(xt_hbm.at[nxt], g_v, isem)

    d_idx.wait()

    @pl.when(wid < N_CONT)
    def _cont():
        fo = N_CAT * EDIM + wid
        s, q = _stats(load_block)
        scale, shift = _scale_shift(s, q, gamma_v, beta_v, fo)
        norm_inplace(scale, shift)
        pltpu.sync_copy(g_v, out_hbm.at[fo])


def kernel(x_in, tables, bn_gamma, bn_beta):
    tt = tables.transpose(0, 2, 1).reshape(N_CAT * EDIM, VOCAB)
    xt = x_in.T
    out_t = _cat_emb_head(tt, xt, bn_gamma, bn_beta)
    return out_t.T
